# Initial kernel scaffold; baseline (speedup 1.0000x reference)
import jax
import jax.numpy as jnp
from jax.experimental import pallas as pl


def kernel(node_feats, edge_index, edge_feats, params):
    raise NotImplementedError("write your pallas kernel here")


# speedup vs baseline: 2.7184x; 2.7184x over previous
import functools

import jax
import jax.numpy as jnp
from jax import lax
from jax.experimental import pallas as pl
from jax.experimental.pallas import tpu as pltpu
from jax.experimental.pallas import tpu_sc as plsc

N = 100000
E = 1600000
IN_DIM = 16
EMB = 50
HID = 100
NUM_TASK = 112

NC = 2
NS = 16
B = 128
ROWS = E // B
ROWS_PER_SC = ROWS // NC

def _mesh():
  return plsc.VectorSubcoreMesh(
      core_axis_name="c", subcore_axis_name="s", num_cores=NC, num_subcores=NS)


def _tile_range(s):
  per = (ROWS_PER_SC + NS - 1) // NS
  rs = jnp.minimum(s * per, ROWS_PER_SC)
  re = jnp.minimum(rs + per, ROWS_PER_SC)
  return rs, re


def _sc_a_body(src2, dst2, ef, nf, zr2, zr1, ones_h, ef_p, deg_p, nf_p,
               sbuf, dbuf, rows, ones_v, zbuf, acc, deg, sem):
  c = lax.axis_index("c")
  s = lax.axis_index("s")
  base = c * ROWS_PER_SC
  rs, re = _tile_range(s)

  pltpu.sync_copy(ones_h, ones_v)
  @pl.when(s < 10)
  def _():
    pltpu.sync_copy(zr2, acc.at[pl.ds(s * 10000, 10000)])
    pltpu.sync_copy(zr1, zbuf)
    pltpu.sync_copy(zbuf, deg.at[pl.ds(s * 10000, 10000)])

  plsc.subcore_barrier()

  def ph1(r, _):
    rr = base + r
    pltpu.sync_copy(dst2.at[pl.ds(rr, 1)], dbuf)
    pltpu.sync_copy(ef.at[pl.ds(rr * B, B)], rows)
    pltpu.sync_copy(rows, acc.at[dbuf.at[0]], add=True)
    pltpu.sync_copy(ones_v, deg.at[dbuf.at[0]], add=True)
    return 0

  lax.fori_loop(rs, re, ph1, 0)
  plsc.subcore_barrier()
  @pl.when(s < 10)
  def _():
    pltpu.sync_copy(acc.at[pl.ds(s * 10000, 10000)],
                    ef_p.at[c, pl.ds(s * 10000, 10000)])
    pltpu.sync_copy(deg.at[pl.ds(s * 10000, 10000)], zbuf)
    pltpu.sync_copy(zbuf, deg_p.at[pl.ds(c * N + s * 10000, 10000)])

  plsc.subcore_barrier()

  @pl.when(s < 10)
  def _():
    pltpu.sync_copy(zr2, acc.at[pl.ds(s * 10000, 10000)])

  plsc.subcore_barrier()

  def ph2(r, _):
    rr = base + r
    pltpu.sync_copy(src2.at[pl.ds(rr, 1)], sbuf)
    pltpu.sync_copy(dst2.at[pl.ds(rr, 1)], dbuf)
    pltpu.async_copy(nf.at[sbuf.at[0]], rows, sem).wait()
    pltpu.sync_copy(rows, acc.at[dbuf.at[0]], add=True)
    return 0

  lax.fori_loop(rs, re, ph2, 0)
  plsc.subcore_barrier()
  @pl.when(s < 10)
  def _():
    pltpu.sync_copy(acc.at[pl.ds(s * 10000, 10000)],
                    nf_p.at[c, pl.ds(s * 10000, 10000)])


@functools.cache
def _sc_a():
  return pl.kernel(
    _sc_a_body,
    out_type=(
        jax.ShapeDtypeStruct((NC, N, 16), jnp.float32),
        jax.ShapeDtypeStruct((NC * N,), jnp.float32),
        jax.ShapeDtypeStruct((NC, N, 16), jnp.float32),
    ),
    mesh=_mesh(),
    compiler_params=pltpu.CompilerParams(use_tc_tiling_on_sc=False),
    scratch_types=[
        pltpu.VMEM((1, B), jnp.int32),
        pltpu.VMEM((1, B), jnp.int32),
        pltpu.VMEM((B, 16), jnp.float32),
        pltpu.VMEM((B,), jnp.float32),
        pltpu.VMEM((10000,), jnp.float32),
        pltpu.VMEM_SHARED((N, 16), jnp.float32),
        pltpu.VMEM_SHARED((N,), jnp.float32),
        pltpu.SemaphoreType.DMA,
    ],
  )


def _sc_b_body(src2, dst2, t0, t1, t2, t3, zr2, out,
               sbuf, dbuf, rows, acc, sem):
  c = lax.axis_index("c")
  s = lax.axis_index("s")
  base = c * ROWS_PER_SC
  rs, re = _tile_range(s)

  for ph, tab in enumerate((t0, t1, t2, t3)):
    @pl.when(s < 10)
    def _():
      pltpu.sync_copy(zr2, acc.at[pl.ds(s * 10000, 10000)])

    plsc.subcore_barrier()

    def ph_body(r, _, tab=tab):
      rr = base + r
      pltpu.sync_copy(src2.at[pl.ds(rr, 1)], sbuf)
      pltpu.sync_copy(dst2.at[pl.ds(rr, 1)], dbuf)
      pltpu.async_copy(tab.at[sbuf.at[0]], rows, sem).wait()
      pltpu.sync_copy(rows, acc.at[dbuf.at[0]], add=True)
      return 0

    lax.fori_loop(rs, re, ph_body, 0)
    plsc.subcore_barrier()
    @pl.when(s < 10)
    def _():
      pltpu.sync_copy(acc.at[pl.ds(s * 10000, 10000)],
                      out.at[2 * ph + c, pl.ds(s * 10000, 10000)])
    plsc.subcore_barrier()


@functools.cache
def _sc_b():
  return pl.kernel(
    _sc_b_body,
    out_type=jax.ShapeDtypeStruct((8, N, 16), jnp.float32),
    mesh=_mesh(),
    compiler_params=pltpu.CompilerParams(use_tc_tiling_on_sc=False),
    scratch_types=[
        pltpu.VMEM((1, B), jnp.int32),
        pltpu.VMEM((1, B), jnp.int32),
        pltpu.VMEM((B, 16), jnp.float32),
        pltpu.VMEM_SHARED((N, 16), jnp.float32),
        pltpu.SemaphoreType.DMA,
    ],
  )


BLK = 2000
GRID = N // BLK


def _stats_block(x, width):
  s = jnp.sum(x, axis=0)
  s2 = jnp.sum(x * x, axis=0)
  z = jnp.zeros((128 - width,), jnp.float32)
  row0 = jnp.concatenate([s, z])[None]
  row1 = jnp.concatenate([s2, z])[None]
  return jnp.concatenate([row0, row1, jnp.zeros((6, 128), jnp.float32)], axis=0)


def _accum_stats(st_ref, blk):
  i = pl.program_id(0)

  @pl.when(i == 0)
  def _():
    st_ref[...] = blk

  @pl.when(i > 0)
  def _():
    st_ref[...] = st_ref[...] + blk


def _bn_coeffs(st_ref, g, bt, width):
  mu = st_ref[0:1, :width] / N
  ms = st_ref[1:2, :width] / N
  var = ms - mu * mu
  rstd = lax.rsqrt(var + 1e-5)
  a = g * rstd
  b = bt - mu * a
  return a, b


def _tc1_body(nf_ref, nfp_ref, efp_ref, degp_ref, neWe, nebe, neW, neb,
              eeW, eeb, W1, b1, x1_ref, hea_ref, st_ref):
  nf = nf_ref[...]
  nfa = nfp_ref[0] + nfp_ref[1]
  efa = efp_ref[0] + efp_ref[1]
  deg = degp_ref[0] + degp_ref[1]
  inv = 1.0 / jnp.maximum(deg, 1.0)
  hv0 = nf @ neWe[...] + nebe[...]
  hea = efa @ eeW[...] + deg * eeb[...]
  t = nfa @ neW[...] + deg * neb[...] + hea
  x = (hv0 + t * inv) @ W1[...] + b1[...]
  x1_ref[...] = x
  hea_ref[...] = hea
  _accum_stats(st_ref, _stats_block(x, HID))


def _tc2_body(x_ref, st_in, g, bt, W2, b2, x2_ref, st_ref, win, wout):
  a, b = _bn_coeffs(st_in, g[...], bt[...], win)
  y = jnp.maximum(x_ref[...] * a + b, 0.0)
  x2 = y @ W2[...] + b2[...]
  x2_ref[...] = x2
  _accum_stats(st_ref, _stats_block(x2, wout))


def _tc3_body(x_ref, st_in, g, bt, t0_ref, t1_ref, t2_ref, t3_ref):
  a, b = _bn_coeffs(st_in, g[...], bt[...], EMB)
  y = jnp.maximum(x_ref[...] * a + b, 0.0)
  yp = jnp.concatenate([y, jnp.zeros((y.shape[0], 64 - EMB), jnp.float32)],
                       axis=1)
  t0_ref[...] = yp[:, 0:16]
  t1_ref[...] = yp[:, 16:32]
  t2_ref[...] = yp[:, 32:48]
  t3_ref[...] = yp[:, 48:64]


def _tc4_body(t0, t1, t2, t3, aggp_ref, hea_ref, degp_ref, W1e, W1, b1,
              x3_ref, st_ref):
  hv1 = jnp.concatenate([t0[...], t1[...], t2[...], t3[...]], axis=1)[:, :EMB]
  aggs = [aggp_ref[2 * c] + aggp_ref[2 * c + 1] for c in range(4)]
  agg1 = jnp.concatenate(aggs, axis=1)[:, :EMB]
  deg = degp_ref[0] + degp_ref[1]
  inv = 1.0 / jnp.maximum(deg, 1.0)
  hx = (agg1 + hea_ref[...]) * inv
  x3 = hv1 @ W1e[...] + hx @ W1[...] + b1[...]
  x3_ref[...] = x3
  _accum_stats(st_ref, _stats_block(x3, HID))


def _tc6_body(x_ref, st_in, g, bt, oW, ob, out_ref):
  a, b = _bn_coeffs(st_in, g[...], bt[...], EMB)
  y = jnp.maximum(x_ref[...] * a + b, 0.0)
  out_ref[...] = y @ oW[...] + ob[...]


def _rows_spec(width):
  return pl.BlockSpec((BLK, width), lambda i: (i, 0))


def _part_spec(k, width):
  return pl.BlockSpec((k, BLK, width), lambda i: (0, i, 0))


def _full_spec(shape):
  return pl.BlockSpec(shape, lambda i: tuple(0 for _ in shape))


_STATS = jax.ShapeDtypeStruct((8, 128), jnp.float32)
_STATS_SPEC = pl.BlockSpec((8, 128), lambda i: (0, 0))


def kernel(node_feats, edge_index, edge_feats, params):
  p = params
  src = edge_index[0].reshape(ROWS, B)
  dst = edge_index[1].reshape(ROWS, B)

  zr2 = jnp.zeros((10000, 16), jnp.float32)
  zr1 = jnp.zeros((10000,), jnp.float32)
  ones_h = jnp.ones((B,), jnp.float32)

  ef_p, deg_p, nf_p = _sc_a()(src, dst, edge_feats, node_feats, zr2, zr1,
                              ones_h)
  deg_p3 = deg_p.reshape(NC, N, 1)

  e0 = 1.0 + p['l0_eps']
  e1 = 1.0 + p['l1_eps']
  r = lambda v: v.reshape(1, -1)

  x1, hea, st1 = pl.pallas_call(
      _tc1_body,
      grid=(GRID,),
      in_specs=[
          _rows_spec(16), _part_spec(2, 16), _part_spec(2, 16),
          _part_spec(2, 1),
          _full_spec((16, EMB)), _full_spec((1, EMB)),
          _full_spec((16, EMB)), _full_spec((1, EMB)),
          _full_spec((16, EMB)), _full_spec((1, EMB)),
          _full_spec((EMB, HID)), _full_spec((1, HID)),
      ],
      out_specs=[_rows_spec(HID), _rows_spec(EMB), _STATS_SPEC],
      out_shape=[
          jax.ShapeDtypeStruct((N, HID), jnp.float32),
          jax.ShapeDtypeStruct((N, EMB), jnp.float32),
          _STATS,
      ],
  )(node_feats, nf_p, ef_p, deg_p3,
    e0 * p['ne_W'], e0 * r(p['ne_b']), p['ne_W'], r(p['ne_b']),
    p['ee_W'], r(p['ee_b']), p['l0_W1'], r(p['l0_b1']))

  def tc2(x, st, g, bt, W2, b2, win, wout):
    return pl.pallas_call(
        functools.partial(_tc2_body, win=win, wout=wout),
        grid=(GRID,),
        in_specs=[
            _rows_spec(win), _STATS_SPEC,
            _full_spec((1, win)), _full_spec((1, win)),
            _full_spec((win, wout)), _full_spec((1, wout)),
        ],
        out_specs=[_rows_spec(wout), _STATS_SPEC],
        out_shape=[jax.ShapeDtypeStruct((N, wout), jnp.float32), _STATS],
    )(x, st, r(g), r(bt), W2, b2)

  x2, st2 = tc2(x1, st1, p['l0_g1'], p['l0_bt1'], p['l0_W2'], r(p['l0_b2']),
                HID, EMB)

  t0, t1, t2, t3 = pl.pallas_call(
      _tc3_body,
      grid=(GRID,),
      in_specs=[_rows_spec(EMB), _STATS_SPEC,
                _full_spec((1, EMB)), _full_spec((1, EMB))],
      out_specs=[_rows_spec(16)] * 4,
      out_shape=[jax.ShapeDtypeStruct((N, 16), jnp.float32)] * 4,
  )(x2, st2, r(p['l0_g2']), r(p['l0_bt2']))

  aggp = _sc_b()(src, dst, t0, t1, t2, t3, zr2)

  x3, st3 = pl.pallas_call(
      _tc4_body,
      grid=(GRID,),
      in_specs=[
          _rows_spec(16), _rows_spec(16), _rows_spec(16), _rows_spec(16),
          _part_spec(8, 16), _rows_spec(EMB), _part_spec(2, 1),
          _full_spec((EMB, HID)), _full_spec((EMB, HID)), _full_spec((1, HID)),
      ],
      out_specs=[_rows_spec(HID), _STATS_SPEC],
      out_shape=[jax.ShapeDtypeStruct((N, HID), jnp.float32), _STATS],
  )(t0, t1, t2, t3, aggp, hea, deg_p3,
    e1 * p['l1_W1'], p['l1_W1'], r(p['l1_b1']))

  x4, st4 = tc2(x3, st3, p['l1_g1'], p['l1_bt1'], p['l1_W2'], r(p['l1_b2']),
                HID, EMB)

  out = pl.pallas_call(
      _tc6_body,
      grid=(GRID,),
      in_specs=[_rows_spec(EMB), _STATS_SPEC,
                _full_spec((1, EMB)), _full_spec((1, EMB)),
                _full_spec((EMB, NUM_TASK)), _full_spec((1, NUM_TASK))],
      out_specs=[_rows_spec(NUM_TASK)],
      out_shape=[jax.ShapeDtypeStruct((N, NUM_TASK), jnp.float32)],
  )(x4, st4, r(p['l1_g2']), r(p['l1_bt2']), p['out_W'], r(p['out_b']))[0]

  return out


# speedup vs baseline: 4.7926x; 1.7630x over previous
import functools

import jax
import jax.numpy as jnp
from jax import lax
from jax.experimental import pallas as pl
from jax.experimental.pallas import tpu as pltpu
from jax.experimental.pallas import tpu_sc as plsc

N = 100000
E = 1600000
IN_DIM = 16
EMB = 50
HID = 100
NUM_TASK = 112

NC = 2
NS = 16
B = 128
ROWS = E // B
ROWS_PER_SC = ROWS // NC

def _mesh():
  return plsc.VectorSubcoreMesh(
      core_axis_name="c", subcore_axis_name="s", num_cores=NC, num_subcores=NS)


def _tile_range(s):
  per = (ROWS_PER_SC + NS - 1) // NS
  rs = jnp.minimum(s * per, ROWS_PER_SC)
  re = jnp.minimum(rs + per, ROWS_PER_SC)
  return rs, re


K = 4


def _gather_scatter_loop(tab, src2, dst2, acc, sbuf, dbuf, rows, sem, sem2,
                         base, rs, re):
  nb = (re - rs) // K

  def macro(m, _):
    r0 = base + rs + m * K
    pltpu.sync_copy(src2.at[pl.ds(r0, K)], sbuf)
    pltpu.sync_copy(dst2.at[pl.ds(r0, K)], dbuf)
    gd = [pltpu.async_copy(tab.at[sbuf.at[j]], rows.at[j], sem)
          for j in range(K)]
    sd = []
    for j in range(K):
      gd[j].wait()
      sd.append(pltpu.async_copy(rows.at[j], acc.at[dbuf.at[j]], sem2,
                                 add=True))
    for d in sd:
      d.wait()
    return 0

  lax.fori_loop(0, nb, macro, 0)

  def tail(r, _):
    rr = base + r
    pltpu.sync_copy(src2.at[pl.ds(rr, 1)], sbuf.at[pl.ds(0, 1)])
    pltpu.sync_copy(dst2.at[pl.ds(rr, 1)], dbuf.at[pl.ds(0, 1)])
    pltpu.async_copy(tab.at[sbuf.at[0]], rows.at[0], sem).wait()
    pltpu.sync_copy(rows.at[0], acc.at[dbuf.at[0]], add=True)
    return 0

  lax.fori_loop(rs + nb * K, re, tail, 0)


def _sc_a_body(src2, dst2, ef, nf, zr2, zr1, ones_h, ef_p, deg_p, nf_p,
               sbuf, dbuf, rows, ones_v, zbuf, acc, deg, sem, sem2):
  c = lax.axis_index("c")
  s = lax.axis_index("s")
  base = c * ROWS_PER_SC
  rs, re = _tile_range(s)

  pltpu.sync_copy(ones_h, ones_v)
  @pl.when(s < 10)
  def _():
    pltpu.sync_copy(zr2, acc.at[pl.ds(s * 10000, 10000)])
    pltpu.sync_copy(zr1, zbuf)
    pltpu.sync_copy(zbuf, deg.at[pl.ds(s * 10000, 10000)])

  plsc.subcore_barrier()

  nb = (re - rs) // K

  def ph1_macro(m, _):
    r0 = base + rs + m * K
    pltpu.sync_copy(dst2.at[pl.ds(r0, K)], dbuf)
    pltpu.sync_copy(ef.at[pl.ds(r0, K)], rows)
    sd = []
    for j in range(K):
      sd.append(pltpu.async_copy(rows.at[j], acc.at[dbuf.at[j]], sem2,
                                 add=True))
      sd.append(pltpu.async_copy(ones_v, deg.at[dbuf.at[j]], sem2, add=True))
    for d in sd:
      d.wait()
    return 0

  lax.fori_loop(0, nb, ph1_macro, 0)

  def ph1_tail(r, _):
    rr = base + r
    pltpu.sync_copy(dst2.at[pl.ds(rr, 1)], dbuf.at[pl.ds(0, 1)])
    pltpu.sync_copy(ef.at[pl.ds(rr, 1)], rows.at[pl.ds(0, 1)])
    pltpu.sync_copy(rows.at[0], acc.at[dbuf.at[0]], add=True)
    pltpu.sync_copy(ones_v, deg.at[dbuf.at[0]], add=True)
    return 0

  lax.fori_loop(rs + nb * K, re, ph1_tail, 0)
  plsc.subcore_barrier()
  @pl.when(s < 10)
  def _():
    pltpu.sync_copy(acc.at[pl.ds(s * 10000, 10000)],
                    ef_p.at[c, pl.ds(s * 10000, 10000)])
    pltpu.sync_copy(deg.at[pl.ds(s * 10000, 10000)], zbuf)
    pltpu.sync_copy(zbuf, deg_p.at[pl.ds(c * N + s * 10000, 10000)])

  plsc.subcore_barrier()

  @pl.when(s < 10)
  def _():
    pltpu.sync_copy(zr2, acc.at[pl.ds(s * 10000, 10000)])

  plsc.subcore_barrier()

  _gather_scatter_loop(nf, src2, dst2, acc, sbuf, dbuf, rows, sem, sem2,
                       base, rs, re)
  plsc.subcore_barrier()
  @pl.when(s < 10)
  def _():
    pltpu.sync_copy(acc.at[pl.ds(s * 10000, 10000)],
                    nf_p.at[c, pl.ds(s * 10000, 10000)])


@functools.cache
def _sc_a():
  return pl.kernel(
    _sc_a_body,
    out_type=(
        jax.ShapeDtypeStruct((NC, N, 16), jnp.float32),
        jax.ShapeDtypeStruct((NC * N,), jnp.float32),
        jax.ShapeDtypeStruct((NC, N, 16), jnp.float32),
    ),
    mesh=_mesh(),
    compiler_params=pltpu.CompilerParams(use_tc_tiling_on_sc=False),
    scratch_types=[
        pltpu.VMEM((K, B), jnp.int32),
        pltpu.VMEM((K, B), jnp.int32),
        pltpu.VMEM((K, B, 16), jnp.float32),
        pltpu.VMEM((B,), jnp.float32),
        pltpu.VMEM((10000,), jnp.float32),
        pltpu.VMEM_SHARED((N, 16), jnp.float32),
        pltpu.VMEM_SHARED((N,), jnp.float32),
        pltpu.SemaphoreType.DMA,
        pltpu.SemaphoreType.DMA,
    ],
  )


def _sc_b_body(src2, dst2, t0, t1, t2, t3, zr2, out,
               sbuf, dbuf, rows, acc, sem, sem2):
  c = lax.axis_index("c")
  s = lax.axis_index("s")
  base = c * ROWS_PER_SC
  rs, re = _tile_range(s)

  for ph, tab in enumerate((t0, t1, t2, t3)):
    @pl.when(s < 10)
    def _():
      pltpu.sync_copy(zr2, acc.at[pl.ds(s * 10000, 10000)])

    plsc.subcore_barrier()

    _gather_scatter_loop(tab, src2, dst2, acc, sbuf, dbuf, rows, sem, sem2,
                         base, rs, re)
    plsc.subcore_barrier()
    @pl.when(s < 10)
    def _():
      pltpu.sync_copy(acc.at[pl.ds(s * 10000, 10000)],
                      out.at[2 * ph + c, pl.ds(s * 10000, 10000)])
    plsc.subcore_barrier()


@functools.cache
def _sc_b():
  return pl.kernel(
    _sc_b_body,
    out_type=jax.ShapeDtypeStruct((8, N, 16), jnp.float32),
    mesh=_mesh(),
    compiler_params=pltpu.CompilerParams(use_tc_tiling_on_sc=False),
    scratch_types=[
        pltpu.VMEM((K, B), jnp.int32),
        pltpu.VMEM((K, B), jnp.int32),
        pltpu.VMEM((K, B, 16), jnp.float32),
        pltpu.VMEM_SHARED((N, 16), jnp.float32),
        pltpu.SemaphoreType.DMA,
        pltpu.SemaphoreType.DMA,
    ],
  )


BLK = 2000
GRID = N // BLK


def _stats_block(x, width):
  s = jnp.sum(x, axis=0)
  s2 = jnp.sum(x * x, axis=0)
  z = jnp.zeros((128 - width,), jnp.float32)
  row0 = jnp.concatenate([s, z])[None]
  row1 = jnp.concatenate([s2, z])[None]
  return jnp.concatenate([row0, row1, jnp.zeros((6, 128), jnp.float32)], axis=0)


def _accum_stats(st_ref, blk):
  i = pl.program_id(0)

  @pl.when(i == 0)
  def _():
    st_ref[...] = blk

  @pl.when(i > 0)
  def _():
    st_ref[...] = st_ref[...] + blk


def _bn_coeffs(st_ref, g, bt, width):
  mu = st_ref[0:1, :width] / N
  ms = st_ref[1:2, :width] / N
  var = ms - mu * mu
  rstd = lax.rsqrt(var + 1e-5)
  a = g * rstd
  b = bt - mu * a
  return a, b


def _tc1_body(nf_ref, nfp_ref, efp_ref, degp_ref, neWe, nebe, neW, neb,
              eeW, eeb, W1, b1, x1_ref, hea_ref, st_ref):
  nf = nf_ref[...]
  nfa = nfp_ref[0] + nfp_ref[1]
  efa = efp_ref[0] + efp_ref[1]
  deg = degp_ref[0] + degp_ref[1]
  inv = 1.0 / jnp.maximum(deg, 1.0)
  hv0 = nf @ neWe[...] + nebe[...]
  hea = efa @ eeW[...] + deg * eeb[...]
  t = nfa @ neW[...] + deg * neb[...] + hea
  x = (hv0 + t * inv) @ W1[...] + b1[...]
  x1_ref[...] = x
  hea_ref[...] = hea
  _accum_stats(st_ref, _stats_block(x, HID))


def _tc2_body(x_ref, st_in, g, bt, W2, b2, x2_ref, st_ref, win, wout):
  a, b = _bn_coeffs(st_in, g[...], bt[...], win)
  y = jnp.maximum(x_ref[...] * a + b, 0.0)
  x2 = y @ W2[...] + b2[...]
  x2_ref[...] = x2
  _accum_stats(st_ref, _stats_block(x2, wout))


def _tc3_body(x_ref, st_in, g, bt, t0_ref, t1_ref, t2_ref, t3_ref):
  a, b = _bn_coeffs(st_in, g[...], bt[...], EMB)
  y = jnp.maximum(x_ref[...] * a + b, 0.0)
  yp = jnp.concatenate([y, jnp.zeros((y.shape[0], 64 - EMB), jnp.float32)],
                       axis=1)
  t0_ref[...] = yp[:, 0:16]
  t1_ref[...] = yp[:, 16:32]
  t2_ref[...] = yp[:, 32:48]
  t3_ref[...] = yp[:, 48:64]


def _tc4_body(t0, t1, t2, t3, aggp_ref, hea_ref, degp_ref, W1e, W1, b1,
              x3_ref, st_ref):
  hv1 = jnp.concatenate([t0[...], t1[...], t2[...], t3[...]], axis=1)[:, :EMB]
  aggs = [aggp_ref[2 * c] + aggp_ref[2 * c + 1] for c in range(4)]
  agg1 = jnp.concatenate(aggs, axis=1)[:, :EMB]
  deg = degp_ref[0] + degp_ref[1]
  inv = 1.0 / jnp.maximum(deg, 1.0)
  hx = (agg1 + hea_ref[...]) * inv
  x3 = hv1 @ W1e[...] + hx @ W1[...] + b1[...]
  x3_ref[...] = x3
  _accum_stats(st_ref, _stats_block(x3, HID))


def _tc6_body(x_ref, st_in, g, bt, oW, ob, out_ref):
  a, b = _bn_coeffs(st_in, g[...], bt[...], EMB)
  y = jnp.maximum(x_ref[...] * a + b, 0.0)
  out_ref[...] = y @ oW[...] + ob[...]


def _rows_spec(width):
  return pl.BlockSpec((BLK, width), lambda i: (i, 0))


def _part_spec(k, width):
  return pl.BlockSpec((k, BLK, width), lambda i: (0, i, 0))


def _full_spec(shape):
  return pl.BlockSpec(shape, lambda i: tuple(0 for _ in shape))


_STATS = jax.ShapeDtypeStruct((8, 128), jnp.float32)
_STATS_SPEC = pl.BlockSpec((8, 128), lambda i: (0, 0))


def kernel(node_feats, edge_index, edge_feats, params):
  p = params
  src = edge_index[0].reshape(ROWS, B)
  dst = edge_index[1].reshape(ROWS, B)

  zr2 = jnp.zeros((10000, 16), jnp.float32)
  zr1 = jnp.zeros((10000,), jnp.float32)
  ones_h = jnp.ones((B,), jnp.float32)

  ef3 = edge_feats.reshape(ROWS, B, 16)
  ef_p, deg_p, nf_p = _sc_a()(src, dst, ef3, node_feats, zr2, zr1, ones_h)
  deg_p3 = deg_p.reshape(NC, N, 1)

  e0 = 1.0 + p['l0_eps']
  e1 = 1.0 + p['l1_eps']
  r = lambda v: v.reshape(1, -1)

  x1, hea, st1 = pl.pallas_call(
      _tc1_body,
      grid=(GRID,),
      in_specs=[
          _rows_spec(16), _part_spec(2, 16), _part_spec(2, 16),
          _part_spec(2, 1),
          _full_spec((16, EMB)), _full_spec((1, EMB)),
          _full_spec((16, EMB)), _full_spec((1, EMB)),
          _full_spec((16, EMB)), _full_spec((1, EMB)),
          _full_spec((EMB, HID)), _full_spec((1, HID)),
      ],
      out_specs=[_rows_spec(HID), _rows_spec(EMB), _STATS_SPEC],
      out_shape=[
          jax.ShapeDtypeStruct((N, HID), jnp.float32),
          jax.ShapeDtypeStruct((N, EMB), jnp.float32),
          _STATS,
      ],
  )(node_feats, nf_p, ef_p, deg_p3,
    e0 * p['ne_W'], e0 * r(p['ne_b']), p['ne_W'], r(p['ne_b']),
    p['ee_W'], r(p['ee_b']), p['l0_W1'], r(p['l0_b1']))

  def tc2(x, st, g, bt, W2, b2, win, wout):
    return pl.pallas_call(
        functools.partial(_tc2_body, win=win, wout=wout),
        grid=(GRID,),
        in_specs=[
            _rows_spec(win), _STATS_SPEC,
            _full_spec((1, win)), _full_spec((1, win)),
            _full_spec((win, wout)), _full_spec((1, wout)),
        ],
        out_specs=[_rows_spec(wout), _STATS_SPEC],
        out_shape=[jax.ShapeDtypeStruct((N, wout), jnp.float32), _STATS],
    )(x, st, r(g), r(bt), W2, b2)

  x2, st2 = tc2(x1, st1, p['l0_g1'], p['l0_bt1'], p['l0_W2'], r(p['l0_b2']),
                HID, EMB)

  t0, t1, t2, t3 = pl.pallas_call(
      _tc3_body,
      grid=(GRID,),
      in_specs=[_rows_spec(EMB), _STATS_SPEC,
                _full_spec((1, EMB)), _full_spec((1, EMB))],
      out_specs=[_rows_spec(16)] * 4,
      out_shape=[jax.ShapeDtypeStruct((N, 16), jnp.float32)] * 4,
  )(x2, st2, r(p['l0_g2']), r(p['l0_bt2']))

  aggp = _sc_b()(src, dst, t0, t1, t2, t3, zr2)

  x3, st3 = pl.pallas_call(
      _tc4_body,
      grid=(GRID,),
      in_specs=[
          _rows_spec(16), _rows_spec(16), _rows_spec(16), _rows_spec(16),
          _part_spec(8, 16), _rows_spec(EMB), _part_spec(2, 1),
          _full_spec((EMB, HID)), _full_spec((EMB, HID)), _full_spec((1, HID)),
      ],
      out_specs=[_rows_spec(HID), _STATS_SPEC],
      out_shape=[jax.ShapeDtypeStruct((N, HID), jnp.float32), _STATS],
  )(t0, t1, t2, t3, aggp, hea, deg_p3,
    e1 * p['l1_W1'], p['l1_W1'], r(p['l1_b1']))

  x4, st4 = tc2(x3, st3, p['l1_g1'], p['l1_bt1'], p['l1_W2'], r(p['l1_b2']),
                HID, EMB)

  out = pl.pallas_call(
      _tc6_body,
      grid=(GRID,),
      in_specs=[_rows_spec(EMB), _STATS_SPEC,
                _full_spec((1, EMB)), _full_spec((1, EMB)),
                _full_spec((EMB, NUM_TASK)), _full_spec((1, NUM_TASK))],
      out_specs=[_rows_spec(NUM_TASK)],
      out_shape=[jax.ShapeDtypeStruct((N, NUM_TASK), jnp.float32)],
  )(x4, st4, r(p['l1_g2']), r(p['l1_bt2']), p['out_W'], r(p['out_b']))[0]

  return out
